# Initial kernel scaffold; baseline (speedup 1.0000x reference)
import jax
import jax.numpy as jnp
from jax.experimental import pallas as pl


def kernel(wavs, w1, b1, w2, b2, w3, b3, w4, b4, codebook, mel_filters, window):
    raise NotImplementedError("write your pallas kernel here")


# speedup vs baseline: 1.4214x; 1.4214x over previous
import numpy as np
import jax
import jax.numpy as jnp
from jax.experimental import pallas as pl
from jax.experimental.pallas import tpu as pltpu

N_FFT = 400
HOP = 160
N_MELS = 128
EMB = 1024
N_CODES = 6561
N_FRAMES = 600
N_FREQ = 201
FREQ_PAD = 256
CODES_PAD = 6656

_n = np.arange(N_FFT, dtype=np.float64)[:, None]
_k = np.arange(N_FREQ, dtype=np.float64)[None, :]
_ang = 2.0 * np.pi * _n * _k / N_FFT
_DFT_COS = np.zeros((N_FFT, FREQ_PAD), np.float32)
_DFT_SIN = np.zeros((N_FFT, FREQ_PAD), np.float32)
_DFT_COS[:, :N_FREQ] = np.cos(_ang).astype(np.float32)
_DFT_SIN[:, :N_FREQ] = np.sin(_ang).astype(np.float32)

_FRAME_IDX = (np.arange(N_FRAMES)[:, None] * HOP
              + np.arange(N_FFT)[None, :]).astype(np.int32)


def _mm(a, b):
    return jax.lax.dot_general(a, b, (((1,), (0,)), ((), ())),
                               preferred_element_type=jnp.float32)


def _shift_down(x):
    return jnp.concatenate([jnp.zeros((1, x.shape[1]), x.dtype), x[:-1]], 0)


def _shift_up(x):
    return jnp.concatenate([x[1:], jnp.zeros((1, x.shape[1]), x.dtype)], 0)


def _pipeline_kernel(frames_ref, cw_ref, sw_ref, melt_ref,
                     w1_ref, b1_ref, w2_ref, b2_ref,
                     w3_ref, b3_ref, w4_ref, b4_ref,
                     cbt_ref, cbn_ref, out_ref):
    fr = frames_ref[0]

    re = _mm(fr, cw_ref[...])
    im = _mm(fr, sw_ref[...])
    mag = re * re + im * im
    mel = _mm(mag, melt_ref[...])
    lg = jnp.log(jnp.maximum(mel, 1e-10)) * np.float32(1.0 / np.log(10.0))
    mx = jnp.max(lg)
    mels = (jnp.maximum(lg, mx - 8.0) + 4.0) * 0.25

    w1 = w1_ref[...]
    h = (_mm(_shift_down(mels), w1[0:128])
         + _mm(mels, w1[128:256])
         + _mm(_shift_up(mels), w1[256:384])
         + b1_ref[...])
    h = jnp.maximum(h, 0.0)

    hr = h.reshape(300, 2, 512)
    even, odd = hr[:, 0, :], hr[:, 1, :]
    w2 = w2_ref[...]
    h = (_mm(_shift_down(odd), w2[0:512])
         + _mm(even, w2[512:1024])
         + _mm(odd, w2[1024:1536])
         + b2_ref[...])
    h = jnp.maximum(h, 0.0)

    hr = h.reshape(150, 2, 512)
    even, odd = hr[:, 0, :], hr[:, 1, :]
    w3 = w3_ref[...]
    h = (_mm(_shift_down(odd), w3[0:512])
         + _mm(even, w3[512:1024])
         + _mm(odd, w3[1024:1536])
         + b3_ref[...])
    h = jnp.maximum(h, 0.0)

    w4 = w4_ref[...]
    enc = (_mm(_shift_down(h), w4[0:1024])
           + _mm(h, w4[1024:2048])
           + _mm(_shift_up(h), w4[2048:3072])
           + b4_ref[...])

    en = jnp.sum(enc * enc, axis=1, keepdims=True)
    d2 = en + cbn_ref[...] - 2.0 * _mm(enc, cbt_ref[...])
    dist = jnp.sqrt(jnp.maximum(d2, 1e-12))
    m = jnp.min(dist, axis=1, keepdims=True)
    ids = jax.lax.broadcasted_iota(jnp.int32, dist.shape, 1)
    am = jnp.min(jnp.where(dist <= m, ids, jnp.int32(2**31 - 1)), axis=1)
    out_ref[0, 0, :] = am


def kernel(wavs, w1, b1, w2, b2, w3, b3, w4, b4, codebook, mel_filters, window):
    B = wavs.shape[0]
    pad = N_FFT // 2
    x = jnp.pad(wavs, ((0, 0), (pad, pad)), mode="reflect")
    frames = x[:, _FRAME_IDX]

    wcol = window[:, None]
    cw = jnp.asarray(_DFT_COS) * wcol
    sw = jnp.asarray(_DFT_SIN) * wcol
    melt = jnp.zeros((FREQ_PAD, N_MELS), jnp.float32).at[:N_FREQ].set(
        mel_filters.T)

    w1r = jnp.transpose(w1, (2, 1, 0)).reshape(3 * N_MELS, 512)
    w2r = jnp.transpose(w2, (2, 1, 0)).reshape(3 * 512, 512)
    w3r = jnp.transpose(w3, (2, 1, 0)).reshape(3 * 512, 1024)
    w4r = jnp.transpose(w4, (2, 1, 0)).reshape(3 * 1024, EMB)

    cbt = jnp.zeros((EMB, CODES_PAD), jnp.float32).at[:, :N_CODES].set(
        codebook.T)
    cbn = jnp.full((1, CODES_PAD), 1e30, jnp.float32).at[0, :N_CODES].set(
        jnp.sum(codebook * codebook, axis=1))

    const = lambda *shape: pl.BlockSpec(shape, lambda b: (0,) * len(shape))
    tokens3 = pl.pallas_call(
        _pipeline_kernel,
        grid=(B,),
        in_specs=[
            pl.BlockSpec((1, N_FRAMES, N_FFT), lambda b: (b, 0, 0)),
            const(N_FFT, FREQ_PAD),
            const(N_FFT, FREQ_PAD),
            const(FREQ_PAD, N_MELS),
            const(3 * N_MELS, 512), const(1, 512),
            const(3 * 512, 512), const(1, 512),
            const(3 * 512, 1024), const(1, 1024),
            const(3 * 1024, EMB), const(1, EMB),
            const(EMB, CODES_PAD),
            const(1, CODES_PAD),
        ],
        out_specs=pl.BlockSpec((1, 1, 150), lambda b: (b, 0, 0)),
        out_shape=jax.ShapeDtypeStruct((B, 1, 150), jnp.int32),
        compiler_params=pltpu.CompilerParams(
            dimension_semantics=("arbitrary",),
            vmem_limit_bytes=120 * 1024 * 1024,
        ),
    )(frames, cw, sw, melt,
      w1r, b1.reshape(1, -1), w2r, b2.reshape(1, -1),
      w3r, b3.reshape(1, -1), w4r, b4.reshape(1, -1), cbt, cbn)

    tokens = tokens3.reshape(B, 150)
    token_lens = jnp.full((B,), 150, dtype=jnp.int32)
    return tokens, token_lens
